# baseline (speedup 1.0000x reference)
import functools

import jax
import jax.numpy as jnp
from jax.experimental import pallas as pl
from jax.experimental.pallas import tpu as pltpu

NUM_RBF = 32
MIN_D = 2.0
MAX_D = 22.0
K = 30
NPAD = 5120
ROWS = 256
PAD_COORD = 1.0e9


def _knn_rbf_kernel(rows_ref, colsT_ref, idx_ref, rbf_ref):
    xr = rows_ref[...]
    ct = colsT_ref[...]
    d2 = ((xr[:, 0:1] - ct[0:1, :]) ** 2
          + (xr[:, 1:2] - ct[1:2, :]) ** 2
          + (xr[:, 2:3] - ct[2:3, :]) ** 2)

    iota = jax.lax.broadcasted_iota(jnp.int32, d2.shape, 1)
    big_idx = jnp.int32(NPAD)
    step = (MAX_D - MIN_D) / (NUM_RBF - 1)
    mu = MIN_D + jax.lax.broadcasted_iota(
        jnp.int32, (1, NUM_RBF), 1).astype(jnp.float32) * step
    sigma = (MAX_D - MIN_D) / NUM_RBF
    inv2s2 = 1.0 / (2.0 * sigma * sigma)

    idx_ref[...] = jnp.zeros((ROWS, 32), jnp.int32)
    vstar = jnp.full((ROWS, 1), -jnp.inf, jnp.float32)
    jstar = jnp.full((ROWS, 1), -1, jnp.int32)
    for k in range(K + 1):
        valid = (d2 > vstar) | ((d2 == vstar) & (iota > jstar))
        masked = jnp.where(valid, d2, jnp.inf)
        m = jnp.min(masked, axis=1, keepdims=True)
        am = jnp.min(jnp.where(masked == m, iota, big_idx),
                     axis=1, keepdims=True)
        vstar, jstar = m, am
        if k == 0:
            continue
        dist = jnp.sqrt(jnp.maximum(m, 1e-12))
        mask = dist <= MAX_D
        dist = jnp.where(mask, dist, 0.0)
        rbf_k = jnp.exp(-((dist - mu) ** 2) * inv2s2)
        rbf_k = rbf_k * mask.astype(rbf_k.dtype)
        idx_ref[:, k - 1:k] = am
        rbf_ref[:, k - 1, :] = rbf_k


@jax.jit
def kernel(coords):
    n = coords.shape[0]
    cpad = jnp.concatenate(
        [coords, jnp.full((NPAD - n, 3), PAD_COORD, coords.dtype)], axis=0)
    rows = jnp.zeros((NPAD, 128), jnp.float32).at[:, :3].set(cpad)
    colsT = jnp.zeros((8, NPAD), jnp.float32).at[:3, :].set(cpad.T)

    idx_out, rbf = pl.pallas_call(
        _knn_rbf_kernel,
        grid=(NPAD // ROWS,),
        in_specs=[
            pl.BlockSpec((ROWS, 128), lambda i: (i, 0)),
            pl.BlockSpec((8, NPAD), lambda i: (0, 0)),
        ],
        out_specs=[
            pl.BlockSpec((ROWS, 32), lambda i: (i, 0)),
            pl.BlockSpec((ROWS, K, NUM_RBF), lambda i: (i, 0, 0)),
        ],
        out_shape=[
            jax.ShapeDtypeStruct((NPAD, 32), jnp.int32),
            jax.ShapeDtypeStruct((NPAD, K, NUM_RBF), jnp.float32),
        ],
        compiler_params=pltpu.CompilerParams(
            dimension_semantics=("parallel",),
        ),
    )(rows, colsT)

    src = jnp.repeat(jnp.arange(n, dtype=jnp.int32), K)
    dst = idx_out[:n, :K].reshape(-1)
    edge_index = jnp.stack([src, dst], axis=0)
    return edge_index, rbf[:n].reshape(n * K, NUM_RBF)


# speedup vs baseline: 1.2115x; 1.2115x over previous
import functools

import jax
import jax.numpy as jnp
from jax.experimental import pallas as pl
from jax.experimental.pallas import tpu as pltpu

NUM_RBF = 32
MIN_D = 2.0
MAX_D = 22.0
K = 30
NPAD = 5120
ROWS = 256
PAD_COORD = 1.0e9


def _knn_rbf_kernel(rows_ref, colsT_ref, idx_ref, rbf_ref):
    xr = rows_ref[...]
    ct = colsT_ref[...]
    d2 = ((xr[:, 0:1] - ct[0:1, :]) ** 2
          + (xr[:, 1:2] - ct[1:2, :]) ** 2
          + (xr[:, 2:3] - ct[2:3, :]) ** 2)

    iota = jax.lax.broadcasted_iota(jnp.int32, d2.shape, 1)
    big_idx = jnp.int32(NPAD)
    step = (MAX_D - MIN_D) / (NUM_RBF - 1)
    mu = MIN_D + jax.lax.broadcasted_iota(
        jnp.int32, (1, NUM_RBF), 1).astype(jnp.float32) * step
    sigma = (MAX_D - MIN_D) / NUM_RBF
    inv2s2 = 1.0 / (2.0 * sigma * sigma)

    idx_ref[...] = jnp.zeros((ROWS, 32), jnp.int32)
    work = d2
    for k in range(K + 1):
        m = jnp.min(work, axis=1, keepdims=True)
        am = jnp.argmin(work, axis=1, keepdims=True).astype(jnp.int32)
        if k < K:
            work = jnp.where(iota == am, jnp.inf, work)
        if k == 0:
            continue
        dist = jnp.sqrt(jnp.maximum(m, 1e-12))
        mask = dist <= MAX_D
        dist = jnp.where(mask, dist, 0.0)
        rbf_k = jnp.exp(-((dist - mu) ** 2) * inv2s2)
        rbf_k = rbf_k * mask.astype(rbf_k.dtype)
        idx_ref[:, k - 1:k] = am
        rbf_ref[:, k - 1, :] = rbf_k


@jax.jit
def kernel(coords):
    n = coords.shape[0]
    cpad = jnp.concatenate(
        [coords, jnp.full((NPAD - n, 3), PAD_COORD, coords.dtype)], axis=0)
    rows = jnp.zeros((NPAD, 128), jnp.float32).at[:, :3].set(cpad)
    colsT = jnp.zeros((8, NPAD), jnp.float32).at[:3, :].set(cpad.T)

    idx_out, rbf = pl.pallas_call(
        _knn_rbf_kernel,
        grid=(NPAD // ROWS,),
        in_specs=[
            pl.BlockSpec((ROWS, 128), lambda i: (i, 0)),
            pl.BlockSpec((8, NPAD), lambda i: (0, 0)),
        ],
        out_specs=[
            pl.BlockSpec((ROWS, 32), lambda i: (i, 0)),
            pl.BlockSpec((ROWS, K, NUM_RBF), lambda i: (i, 0, 0)),
        ],
        out_shape=[
            jax.ShapeDtypeStruct((NPAD, 32), jnp.int32),
            jax.ShapeDtypeStruct((NPAD, K, NUM_RBF), jnp.float32),
        ],
        compiler_params=pltpu.CompilerParams(
            dimension_semantics=("parallel",),
        ),
    )(rows, colsT)

    src = jnp.repeat(jnp.arange(n, dtype=jnp.int32), K)
    dst = idx_out[:n, :K].reshape(-1)
    edge_index = jnp.stack([src, dst], axis=0)
    return edge_index, rbf[:n].reshape(n * K, NUM_RBF)


# speedup vs baseline: 1.4320x; 1.1820x over previous
import jax
import jax.numpy as jnp
from jax.experimental import pallas as pl
from jax.experimental.pallas import tpu as pltpu

NUM_RBF = 32
MIN_D = 2.0
MAX_D = 22.0
K = 30
NPAD = 5120
ROWS = 256
FOLD = 4
W = NPAD // FOLD
PAD_COORD = 1.0e9


def _cmp_exchange(va, ia, vb, ib):
    swap = (va > vb) | ((va == vb) & (ia > ib))
    return (jnp.where(swap, vb, va), jnp.where(swap, ib, ia),
            jnp.where(swap, va, vb), jnp.where(swap, ia, ib))


def _knn_rbf_kernel(rows_ref, colsT_ref, idx_ref, rbf_ref):
    xr = rows_ref[...]
    ct = colsT_ref[...]

    iota = jax.lax.broadcasted_iota(jnp.int32, (ROWS, W), 1)
    vs = []
    ids = []
    for q in range(FOLD):
        sl = slice(q * W, (q + 1) * W)
        d2q = ((xr[:, 0:1] - ct[0:1, sl]) ** 2
               + (xr[:, 1:2] - ct[1:2, sl]) ** 2
               + (xr[:, 2:3] - ct[2:3, sl]) ** 2)
        vs.append(jnp.sqrt(jnp.maximum(d2q, 1e-12)))
        ids.append(iota * FOLD + q)

    v0, i0, v1, i1 = vs[0], ids[0], vs[1], ids[1]
    v2, i2, v3, i3 = vs[2], ids[2], vs[3], ids[3]
    v0, i0, v1, i1 = _cmp_exchange(v0, i0, v1, i1)
    v2, i2, v3, i3 = _cmp_exchange(v2, i2, v3, i3)
    v0, i0, v2, i2 = _cmp_exchange(v0, i0, v2, i2)
    v1, i1, v3, i3 = _cmp_exchange(v1, i1, v3, i3)
    v1, i1, v2, i2 = _cmp_exchange(v1, i1, v2, i2)

    big_idx = jnp.int32(NPAD)
    inf = jnp.float32(jnp.inf)
    step = (MAX_D - MIN_D) / (NUM_RBF - 1)
    mu = MIN_D + jax.lax.broadcasted_iota(
        jnp.int32, (1, NUM_RBF), 1).astype(jnp.float32) * step
    sigma = (MAX_D - MIN_D) / NUM_RBF
    inv2s2 = 1.0 / (2.0 * sigma * sigma)

    idx_ref[...] = jnp.zeros((ROWS, 32), jnp.int32)
    for k in range(K + 1):
        m = jnp.min(v0, axis=1, keepdims=True)
        am = jnp.min(jnp.where(v0 == m, i0, big_idx),
                     axis=1, keepdims=True)
        if k < K:
            upd = i0 == am
            v0 = jnp.where(upd, v1, v0)
            i0 = jnp.where(upd, i1, i0)
            v1 = jnp.where(upd, v2, v1)
            i1 = jnp.where(upd, i2, i1)
            v2 = jnp.where(upd, v3, v2)
            i2 = jnp.where(upd, i3, i2)
            v3 = jnp.where(upd, inf, v3)
        if k == 0:
            continue
        mask = m <= MAX_D
        dist = jnp.where(mask, m, 0.0)
        rbf_k = jnp.exp(-((dist - mu) ** 2) * inv2s2)
        rbf_k = rbf_k * mask.astype(rbf_k.dtype)
        idx_ref[:, k - 1:k] = am
        rbf_ref[:, k - 1, :] = rbf_k


@jax.jit
def kernel(coords):
    n = coords.shape[0]
    cpad = jnp.concatenate(
        [coords, jnp.full((NPAD - n, 3), PAD_COORD, coords.dtype)], axis=0)
    rows = jnp.zeros((NPAD, 128), jnp.float32).at[:, :3].set(cpad)
    cperm = cpad.reshape(W, FOLD, 3).transpose(1, 0, 2).reshape(NPAD, 3)
    colsT = jnp.zeros((8, NPAD), jnp.float32).at[:3, :].set(cperm.T)

    idx_out, rbf = pl.pallas_call(
        _knn_rbf_kernel,
        grid=(NPAD // ROWS,),
        in_specs=[
            pl.BlockSpec((ROWS, 128), lambda i: (i, 0)),
            pl.BlockSpec((8, NPAD), lambda i: (0, 0)),
        ],
        out_specs=[
            pl.BlockSpec((ROWS, 32), lambda i: (i, 0)),
            pl.BlockSpec((ROWS, K, NUM_RBF), lambda i: (i, 0, 0)),
        ],
        out_shape=[
            jax.ShapeDtypeStruct((NPAD, 32), jnp.int32),
            jax.ShapeDtypeStruct((NPAD, K, NUM_RBF), jnp.float32),
        ],
        compiler_params=pltpu.CompilerParams(
            dimension_semantics=("parallel",),
        ),
    )(rows, colsT)

    src = jnp.repeat(jnp.arange(n, dtype=jnp.int32), K)
    dst = idx_out[:n, :K].reshape(-1)
    edge_index = jnp.stack([src, dst], axis=0)
    return edge_index, rbf[:n].reshape(n * K, NUM_RBF)


# speedup vs baseline: 1.5774x; 1.1015x over previous
import jax
import jax.numpy as jnp
from jax.experimental import pallas as pl
from jax.experimental.pallas import tpu as pltpu

NUM_RBF = 32
MIN_D = 2.0
MAX_D = 22.0
K = 30
NPAD = 5120
ROWS = 512
FOLD = 4
W = NPAD // FOLD
PAD_COORD = 1.0e9


def _cmp_exchange(va, ia, vb, ib):
    swap = (va > vb) | ((va == vb) & (ia > ib))
    return (jnp.where(swap, vb, va), jnp.where(swap, ib, ia),
            jnp.where(swap, va, vb), jnp.where(swap, ia, ib))


def _knn_rbf_kernel(rows_ref, colsT_ref, idx_ref, rbf_ref):
    xr = rows_ref[...]
    ct = colsT_ref[...]

    iota = jax.lax.broadcasted_iota(jnp.int32, (ROWS, W), 1)
    vs = []
    ids = []
    for q in range(FOLD):
        sl = slice(q * W, (q + 1) * W)
        d2q = ((xr[:, 0:1] - ct[0:1, sl]) ** 2
               + (xr[:, 1:2] - ct[1:2, sl]) ** 2
               + (xr[:, 2:3] - ct[2:3, sl]) ** 2)
        vs.append(jnp.sqrt(jnp.maximum(d2q, 1e-12)))
        ids.append(iota * FOLD + q)

    v0, i0, v1, i1 = vs[0], ids[0], vs[1], ids[1]
    v2, i2, v3, i3 = vs[2], ids[2], vs[3], ids[3]
    v0, i0, v1, i1 = _cmp_exchange(v0, i0, v1, i1)
    v2, i2, v3, i3 = _cmp_exchange(v2, i2, v3, i3)
    v0, i0, v2, i2 = _cmp_exchange(v0, i0, v2, i2)
    v1, i1, v3, i3 = _cmp_exchange(v1, i1, v3, i3)
    v1, i1, v2, i2 = _cmp_exchange(v1, i1, v2, i2)

    big_idx = jnp.int32(NPAD)
    inf = jnp.float32(jnp.inf)
    step = (MAX_D - MIN_D) / (NUM_RBF - 1)
    mu = MIN_D + jax.lax.broadcasted_iota(
        jnp.int32, (1, NUM_RBF), 1).astype(jnp.float32) * step
    sigma = (MAX_D - MIN_D) / NUM_RBF
    inv2s2 = 1.0 / (2.0 * sigma * sigma)

    idx_ref[...] = jnp.zeros((ROWS, 32), jnp.int32)
    for k in range(K + 1):
        m = jnp.min(v0, axis=1, keepdims=True)
        am = jnp.min(jnp.where(v0 == m, i0, big_idx),
                     axis=1, keepdims=True)
        if k < K:
            upd = i0 == am
            v0 = jnp.where(upd, v1, v0)
            i0 = jnp.where(upd, i1, i0)
            v1 = jnp.where(upd, v2, v1)
            i1 = jnp.where(upd, i2, i1)
            v2 = jnp.where(upd, v3, v2)
            i2 = jnp.where(upd, i3, i2)
            v3 = jnp.where(upd, inf, v3)
        if k == 0:
            continue
        mask = m <= MAX_D
        dist = jnp.where(mask, m, 0.0)
        rbf_k = jnp.exp(-((dist - mu) ** 2) * inv2s2)
        rbf_k = rbf_k * mask.astype(rbf_k.dtype)
        idx_ref[:, k - 1:k] = am
        rbf_ref[:, k - 1, :] = rbf_k


@jax.jit
def kernel(coords):
    n = coords.shape[0]
    cpad = jnp.concatenate(
        [coords, jnp.full((NPAD - n, 3), PAD_COORD, coords.dtype)], axis=0)
    rows = jnp.zeros((NPAD, 128), jnp.float32).at[:, :3].set(cpad)
    cperm = cpad.reshape(W, FOLD, 3).transpose(1, 0, 2).reshape(NPAD, 3)
    colsT = jnp.zeros((8, NPAD), jnp.float32).at[:3, :].set(cperm.T)

    idx_out, rbf = pl.pallas_call(
        _knn_rbf_kernel,
        grid=(NPAD // ROWS,),
        in_specs=[
            pl.BlockSpec((ROWS, 128), lambda i: (i, 0)),
            pl.BlockSpec((8, NPAD), lambda i: (0, 0)),
        ],
        out_specs=[
            pl.BlockSpec((ROWS, 32), lambda i: (i, 0)),
            pl.BlockSpec((ROWS, K, NUM_RBF), lambda i: (i, 0, 0)),
        ],
        out_shape=[
            jax.ShapeDtypeStruct((NPAD, 32), jnp.int32),
            jax.ShapeDtypeStruct((NPAD, K, NUM_RBF), jnp.float32),
        ],
        compiler_params=pltpu.CompilerParams(
            dimension_semantics=("parallel",),
        ),
    )(rows, colsT)

    src = jnp.repeat(jnp.arange(n, dtype=jnp.int32), K)
    dst = idx_out[:n, :K].reshape(-1)
    edge_index = jnp.stack([src, dst], axis=0)
    return edge_index, rbf[:n].reshape(n * K, NUM_RBF)
